# baseline (speedup 1.0000x reference)
import functools

import jax
import jax.numpy as jnp
from jax import lax
from jax.experimental import pallas as pl
from jax.experimental.pallas import tpu as pltpu
from jax.experimental.pallas import tpu_sc as plsc

VOCAB = 1000000
D = 32
MAXLEN = 200
B = 4096

GPR = 64
SROWS = VOCAB // GPR
MBLK = 488
NBLK = 33
RMAT = NBLK * MBLK
VPAD = RMAT * GPR

NC = 2
NS = 16
NW = NC * NS
NTOK = B * MAXLEN
ROWS_PER_W = NTOK // NW
DMA_ROWS = 128
NDMA = ROWS_PER_W // DMA_ROWS
PH = 5
KP = NDMA // PH


def _matmul_body(s_ref, w_ref, b_ref, o_ref):
    o_ref[:] = (
        jnp.dot(s_ref[:], w_ref[:], preferred_element_type=jnp.float32)
        + b_ref[:]
    )


def _table_times_w(S5, W128, b128):
    return pl.pallas_call(
        _matmul_body,
        grid=(NBLK,),
        in_specs=[
            pl.BlockSpec((MBLK, GPR * D), lambda i: (i, 0)),
            pl.BlockSpec((GPR * D, 128), lambda i: (0, 0)),
            pl.BlockSpec((1, 128), lambda i: (0, 0)),
        ],
        out_specs=pl.BlockSpec((MBLK, 128), lambda i: (i, 0)),
        out_shape=jax.ShapeDtypeStruct((RMAT, 128), jnp.float32),
    )(S5, W128, b128)


def _gather_body(sw_hbm, idx_hbm, out_hbm, idx_v, buf0, buf1, sg0, sg1, so):
    wid = lax.axis_index("s") * NC + lax.axis_index("c")
    pltpu.sync_copy(idx_hbm.at[wid], idx_v)
    bufs = (buf0, buf1)
    sems = (sg0, sg1)

    def fire_phase(p, buf, sem):
        def fire(j, c):
            pltpu.make_async_copy(
                sw_hbm.at[idx_v.at[p * KP + j]], buf.at[j], sem
            ).start()
            return c

        lax.fori_loop(0, KP, fire, 0)

    def drain_phase(buf, sem):
        def dr(j, c):
            pltpu.make_async_copy(
                sw_hbm.at[idx_v.at[0]], buf.at[j], sem
            ).wait()
            return c

        lax.fori_loop(0, KP, dr, 0)

    def out_copy(p, buf):
        return pltpu.make_async_copy(
            buf, out_hbm.at[wid].at[pl.ds(p * KP, KP)], so
        )

    fire_phase(0, buf0, sg0)
    fire_phase(1, buf1, sg1)
    for p in range(PH):
        buf, sem = bufs[p % 2], sems[p % 2]
        drain_phase(buf, sem)
        out_copy(p, buf).start()
        if p + 2 < PH:
            out_copy(p, buf).wait()
            fire_phase(p + 2, buf, sem)
    out_copy(PH - 2, bufs[(PH - 2) % 2]).wait()
    out_copy(PH - 1, bufs[(PH - 1) % 2]).wait()


def _sc_gather(sw, idx3):
    mesh = plsc.VectorSubcoreMesh(core_axis_name="c", subcore_axis_name="s")
    f = functools.partial(
        pl.kernel,
        mesh=mesh,
        out_type=jax.ShapeDtypeStruct((NW, NDMA, DMA_ROWS, 2), jnp.float32),
        scratch_types=[
            pltpu.VMEM((NDMA, DMA_ROWS), jnp.int32),
            pltpu.VMEM((KP, DMA_ROWS, 2), jnp.float32),
            pltpu.VMEM((KP, DMA_ROWS, 2), jnp.float32),
            pltpu.SemaphoreType.DMA,
            pltpu.SemaphoreType.DMA,
            pltpu.SemaphoreType.DMA,
        ],
        compiler_params=pltpu.CompilerParams(use_tc_tiling_on_sc=False),
    )(_gather_body)
    return f(sw, idx3)


def _softmax_body(x_ref, o_ref):
    x = x_ref[:]
    m = jnp.max(x, axis=0, keepdims=True)
    e = jnp.exp(x - m)
    o_ref[:] = e / jnp.sum(e, axis=0, keepdims=True)


def _softmax0(x):
    return pl.pallas_call(
        _softmax_body,
        out_shape=jax.ShapeDtypeStruct(x.shape, jnp.float32),
    )(x)


def kernel(inputs, S, T, L, P, W, b):
    idx3 = inputs[0].astype(jnp.int32).reshape(NW, NDMA, DMA_ROWS)
    S5 = S.reshape(SROWS, GPR * D)
    W128 = jnp.kron(jnp.eye(GPR, dtype=jnp.float32), W.astype(jnp.float32))
    b128 = jnp.tile(b.astype(jnp.float32), GPR).reshape(1, 128)
    sw128 = _table_times_w(S5, W128, b128)
    sw2 = sw128.reshape(VPAD, 2)
    logits = _sc_gather(sw2, idx3)
    y = _softmax0(logits.reshape(B, MAXLEN * 2))
    return y.reshape(B, MAXLEN, 2)


# speedup vs baseline: 1.3942x; 1.3942x over previous
import functools

import jax
import jax.numpy as jnp
from jax import lax
from jax.experimental import pallas as pl
from jax.experimental.pallas import tpu as pltpu
from jax.experimental.pallas import tpu_sc as plsc

VOCAB = 1000000
D = 32
MAXLEN = 200
B = 4096

ROW_BLK = 8000

NC = 2
NS = 16
NW = NC * NS
MP = 256
NTOK = B * MAXLEN
TPW = NTOK // NW
IDXW = 2 * TPW + 128
GPH = 32


def _matmul_body(s_ref, w_ref, b_ref, o_ref):
    o_ref[:] = (
        jnp.dot(s_ref[:], w_ref[:], preferred_element_type=jnp.float32)
        + b_ref[:]
    )


def _table_times_w(S, W, b):
    return pl.pallas_call(
        _matmul_body,
        grid=(VOCAB // ROW_BLK,),
        in_specs=[
            pl.BlockSpec((ROW_BLK, D), lambda i: (i, 0)),
            pl.BlockSpec((D, 2), lambda i: (0, 0)),
            pl.BlockSpec((1, 2), lambda i: (0, 0)),
        ],
        out_specs=pl.BlockSpec((ROW_BLK, 2), lambda i: (i, 0)),
        out_shape=jax.ShapeDtypeStruct((VOCAB, 2), jnp.float32),
    )(S, W, b.reshape(1, 2))


def _gather_body(sw_hbm, idx_hbm, out_hbm, idx_v, buf0, buf1, sg0, sg1, so):
    wid = lax.axis_index("s") * NC + lax.axis_index("c")
    pltpu.sync_copy(idx_hbm.at[pl.ds(wid * 2 * TPW, IDXW)], idx_v)
    bufs = (buf0, buf1)
    sems = (sg0, sg1)

    def fire_phase(p, buf, sem):
        def fire(k, c):
            i = k // 4
            g = k % 4
            off = ((p * 4 + i) * MAXLEN + g * 64) * 2
            pltpu.make_async_copy(
                sw_hbm.at[idx_v.at[pl.ds(off, 128)]], buf.at[g, i], sem
            ).start()
            return c

        lax.fori_loop(0, 16, fire, 0)

    def drain_phase(buf, sem):
        def dr(k, c):
            pltpu.make_async_copy(
                sw_hbm.at[idx_v.at[pl.ds(0, 128)]], buf.at[0, 0], sem
            ).wait()
            return c

        lax.fori_loop(0, 16, dr, 0)

    def out_copy(p, buf, g):
        return pltpu.make_async_copy(
            buf.at[g], out_hbm.at[g].at[pl.ds(wid * 128 + p * 4, 4)], so
        )

    fire_phase(0, buf0, sg0)
    fire_phase(1, buf1, sg1)
    for p in range(GPH):
        buf, sem = bufs[p % 2], sems[p % 2]
        drain_phase(buf, sem)
        for g in range(4):
            out_copy(p, buf, g).start()
        if p + 2 < GPH:
            for g in range(4):
                out_copy(p, buf, g).wait()
            fire_phase(p + 2, buf, sem)
    for p in (GPH - 2, GPH - 1):
        for g in range(4):
            out_copy(p, bufs[p % 2], g).wait()


def _sc_gather(sw, idxf):
    mesh = plsc.VectorSubcoreMesh(core_axis_name="c", subcore_axis_name="s")
    f = functools.partial(
        pl.kernel,
        mesh=mesh,
        out_type=jax.ShapeDtypeStruct((4, B, 128), jnp.float32),
        scratch_types=[
            pltpu.VMEM((IDXW,), jnp.int32),
            pltpu.VMEM((4, 4, 128), jnp.float32),
            pltpu.VMEM((4, 4, 128), jnp.float32),
            pltpu.SemaphoreType.DMA,
            pltpu.SemaphoreType.DMA,
            pltpu.SemaphoreType.DMA,
        ],
        compiler_params=pltpu.CompilerParams(use_tc_tiling_on_sc=False),
    )(_gather_body)
    return f(sw, idxf)


def _softmax_body(x_ref, o_ref):
    x = x_ref[0]
    m = jnp.max(x, axis=0, keepdims=True)
    e = jnp.exp(x - m)
    o_ref[0] = e / jnp.sum(e, axis=0, keepdims=True)


def _softmax0(x):
    return pl.pallas_call(
        _softmax_body,
        grid=(4,),
        in_specs=[pl.BlockSpec((1, B, 128), lambda i: (i, 0, 0))],
        out_specs=pl.BlockSpec((1, B, 128), lambda i: (i, 0, 0)),
        out_shape=jax.ShapeDtypeStruct((4, B, 128), jnp.float32),
    )(x)


def kernel(inputs, S, T, L, P, W, b):
    iw = inputs[0].astype(jnp.int32).reshape(NTOK, 1) * 2
    iw = (iw + jnp.array([[0, 1]], jnp.int32)).reshape(2 * NTOK)
    idxf = jnp.concatenate([iw, jnp.zeros(128, jnp.int32)])
    sw = _table_times_w(S, W.astype(jnp.float32), b.astype(jnp.float32))
    g = _sc_gather(sw.reshape(2 * VOCAB), idxf)
    y = _softmax0(g)
    out = y.reshape(4, B, 64, 2).transpose(1, 0, 2, 3).reshape(B, MP, 2)
    return out[:, :MAXLEN, :]
